# baseline (speedup 1.0000x reference)
import jax
import jax.numpy as jnp
from jax import lax
from jax.experimental import pallas as pl
from jax.experimental.pallas import tpu as pltpu

_K = 1024
_BF = jnp.bfloat16


def _dot(a, b):
    return lax.dot_general(a, b, (((1,), (0,)), ((), ())),
                           preferred_element_type=jnp.float32)


def _proj_body(z_ref, w_ref, b_ref, fl_ref):
    ze = _dot(w_ref[...].astype(_BF), z_ref[0].astype(_BF)) + b_ref[...]
    fl_ref[0] = ze.T


def _vq_body(z_ref, w_ref, b_ref, f2_ref, e2_ref, e2c_ref, et_ref,
             zq_ref, ind_ref, diff_ref):
    b = pl.program_id(0)
    nb = pl.num_programs(0)
    hw = z_ref.shape[2]
    ne = e2_ref.shape[0]
    nchunks = ne // _K

    ze = _dot(w_ref[...].astype(_BF), z_ref[0].astype(_BF)) + b_ref[...]
    a_bf = (ze + ze).astype(_BF)
    f2 = f2_ref[0]

    iota = lax.broadcasted_iota(jnp.int32, (_K, hw), 0)

    def scan_chunk(i, carry):
        run_min, run_idx = carry
        e = e2_ref[pl.ds(i * _K, _K), :].astype(_BF)
        mm2 = _dot(e, a_bf)
        d = (f2 - mm2) + e2c_ref[pl.ds(i * _K, _K), :]
        cmin = jnp.min(d, axis=0, keepdims=True)
        cidx = jnp.min(jnp.where(d == cmin, iota, _K),
                       axis=0, keepdims=True) + i * _K
        better = cmin < run_min
        return (jnp.where(better, cmin, run_min),
                jnp.where(better, cidx, run_idx))

    init = (jnp.full((1, hw), jnp.inf, jnp.float32),
            jnp.zeros((1, hw), jnp.int32))
    half = nchunks // 2
    min_a, idx_a = lax.fori_loop(0, half, scan_chunk, init)
    min_b, idx_b = lax.fori_loop(half, nchunks, scan_chunk, init)
    neg_a = (-min_a).astype(_BF).astype(jnp.float32)
    pick_b = (-min_b) > neg_a
    run_idx = jnp.where(pick_b, idx_b, idx_a)
    ind_ref[0] = run_idx

    def zq_chunk(i, acc):
        et = et_ref[:, pl.ds(i * _K, _K)]
        onehot = ((iota + i * _K) == run_idx).astype(jnp.float32)
        return acc + lax.dot_general(et, onehot, (((1,), (0,)), ((), ())),
                                     preferred_element_type=jnp.float32,
                                     precision=lax.Precision.HIGHEST)

    zq = lax.fori_loop(0, nchunks, zq_chunk, jnp.zeros(ze.shape, jnp.float32))
    zq_ref[0] = zq

    sq = jnp.sum(jnp.sum((zq - ze) ** 2, axis=0, keepdims=True),
                 axis=1, keepdims=True)

    @pl.when(b == 0)
    def _init():
        diff_ref[...] = jnp.zeros_like(diff_ref)

    diff_ref[...] += sq

    @pl.when(b == nb - 1)
    def _finish():
        n_elem = nb * hw * zq_ref.shape[1]
        diff_ref[...] = diff_ref[...] * (12.5 / n_elem)


def kernel(z, W_proj, b_proj, embed_w):
    B, C, H, W = z.shape
    D = W_proj.shape[0]
    NE = embed_w.shape[0]
    HW = H * W

    z3 = z.reshape(B, C, HW)
    bp = b_proj.reshape(D, 1)

    flatten = pl.pallas_call(
        _proj_body,
        grid=(B,),
        in_specs=[
            pl.BlockSpec((1, C, HW), lambda b: (b, 0, 0)),
            pl.BlockSpec((D, C), lambda b: (0, 0)),
            pl.BlockSpec((D, 1), lambda b: (0, 0)),
        ],
        out_specs=pl.BlockSpec((1, HW, D), lambda b: (b, 0, 0)),
        out_shape=jax.ShapeDtypeStruct((B, HW, D), jnp.float32),
    )(z3, W_proj, bp)

    sum_f2 = jnp.sum(flatten.reshape(B, H, W, D) ** 2,
                     axis=3).reshape(B, 1, HW)
    sum_e2 = jnp.sum(embed_w ** 2, axis=1, keepdims=True)

    zq, ind, diff = pl.pallas_call(
        _vq_body,
        grid=(B,),
        in_specs=[
            pl.BlockSpec((1, C, HW), lambda b: (b, 0, 0)),
            pl.BlockSpec((D, C), lambda b: (0, 0)),
            pl.BlockSpec((D, 1), lambda b: (0, 0)),
            pl.BlockSpec((1, 1, HW), lambda b: (b, 0, 0)),
            pl.BlockSpec((NE, D), lambda b: (0, 0)),
            pl.BlockSpec((NE, 1), lambda b: (0, 0)),
            pl.BlockSpec((D, NE), lambda b: (0, 0)),
        ],
        out_specs=[
            pl.BlockSpec((1, D, HW), lambda b: (b, 0, 0)),
            pl.BlockSpec((1, 1, HW), lambda b: (b, 0, 0)),
            pl.BlockSpec((1, 1), lambda b: (0, 0)),
        ],
        out_shape=[
            jax.ShapeDtypeStruct((B, D, HW), jnp.float32),
            jax.ShapeDtypeStruct((B, 1, HW), jnp.int32),
            jax.ShapeDtypeStruct((1, 1), jnp.float32),
        ],
    )(z3, W_proj, bp, sum_f2, embed_w, sum_e2, embed_w.T)

    return zq.reshape(B, D, H, W), diff.reshape(()), ind.reshape(B, H, W)


# speedup vs baseline: 2.0533x; 2.0533x over previous
import functools

import jax
import jax.numpy as jnp
from jax import lax
from jax.experimental import pallas as pl
from jax.experimental.pallas import tpu as pltpu
from jax.experimental.pallas import tpu_sc as plsc

_K = 1024
_BF = jnp.bfloat16


def _dot(a, b):
    return lax.dot_general(a, b, (((1,), (0,)), ((), ())),
                           preferred_element_type=jnp.float32)


def _proj_body(z_ref, w_ref, b_ref, fl_ref):
    ze = _dot(w_ref[...].astype(_BF), z_ref[0].astype(_BF)) + b_ref[...]
    fl_ref[0] = ze.T


def _vq_body(z_ref, w_ref, b_ref, f2_ref, e_ref, e2c_ref, ind_ref, diff_ref):
    b = pl.program_id(0)
    nb = pl.num_programs(0)
    hw = z_ref.shape[2]
    ne = e_ref.shape[0]
    nchunks = ne // _K

    ze = _dot(w_ref[...].astype(_BF), z_ref[0].astype(_BF)) + b_ref[...]
    a_bf = (ze + ze).astype(_BF)
    f2 = f2_ref[0]

    iota = lax.broadcasted_iota(jnp.int32, (_K, hw), 0)

    def scan_chunk(i, carry):
        run_min, run_idx = carry
        e = e_ref[pl.ds(i * _K, _K), :].astype(_BF)
        mm2 = _dot(e, a_bf)
        d = (f2 - mm2) + e2c_ref[pl.ds(i * _K, _K), :]
        cmin = jnp.min(d, axis=0, keepdims=True)
        cidx = jnp.min(jnp.where(d == cmin, iota, _K),
                       axis=0, keepdims=True) + i * _K
        better = cmin < run_min
        return (jnp.where(better, cmin, run_min),
                jnp.where(better, cidx, run_idx))

    init = (jnp.full((1, hw), jnp.inf, jnp.float32),
            jnp.zeros((1, hw), jnp.int32))
    half = nchunks // 2
    min_a, idx_a = lax.fori_loop(0, half, scan_chunk, init)
    min_b, idx_b = lax.fori_loop(half, nchunks, scan_chunk, init)
    neg_a = (-min_a).astype(_BF).astype(jnp.float32)
    pick_b = (-min_b) > neg_a
    ind_ref[0] = jnp.where(pick_b, idx_b, idx_a)

    win = jnp.where(pick_b, min_b, min_a)
    sq = jnp.sum(win, axis=1, keepdims=True)

    @pl.when(b == 0)
    def _init():
        diff_ref[...] = jnp.zeros_like(diff_ref)

    diff_ref[...] += sq

    @pl.when(b == nb - 1)
    def _finish():
        n_elem = nb * hw * w_ref.shape[0]
        diff_ref[...] = diff_ref[...] * (12.5 / n_elem)


def _sc_gather(table, idx, n_rows, d):
    info = plsc.get_sparse_core_info()
    nw = info.num_cores * info.num_subcores
    per_w = n_rows // nw
    mesh = plsc.VectorSubcoreMesh(core_axis_name="c", subcore_axis_name="s")

    @functools.partial(
        pl.kernel, mesh=mesh,
        out_type=jax.ShapeDtypeStruct((n_rows, d), jnp.float32),
        scratch_types=[
            pltpu.VMEM((per_w,), jnp.int32),
            pltpu.VMEM((per_w, d), jnp.float32),
            pltpu.SemaphoreType.DMA,
        ],
    )
    def k(table_hbm, idx_hbm, out_hbm, idx_v, rows_v, sem):
        wid = lax.axis_index("s") * info.num_cores + lax.axis_index("c")
        base = wid * per_w
        pltpu.sync_copy(idx_hbm.at[pl.ds(base, per_w)], idx_v)
        pltpu.async_copy(table_hbm.at[idx_v], rows_v, sem).wait()
        pltpu.sync_copy(rows_v, out_hbm.at[pl.ds(base, per_w)])

    return k(table, idx)


def kernel(z, W_proj, b_proj, embed_w):
    B, C, H, W = z.shape
    D = W_proj.shape[0]
    NE = embed_w.shape[0]
    HW = H * W

    z3 = z.reshape(B, C, HW)
    bp = b_proj.reshape(D, 1)

    flatten = pl.pallas_call(
        _proj_body,
        grid=(B,),
        in_specs=[
            pl.BlockSpec((1, C, HW), lambda b: (b, 0, 0)),
            pl.BlockSpec((D, C), lambda b: (0, 0)),
            pl.BlockSpec((D, 1), lambda b: (0, 0)),
        ],
        out_specs=pl.BlockSpec((1, HW, D), lambda b: (b, 0, 0)),
        out_shape=jax.ShapeDtypeStruct((B, HW, D), jnp.float32),
    )(z3, W_proj, bp)

    sum_f2 = jnp.sum(flatten.reshape(B, H, W, D) ** 2,
                     axis=3).reshape(B, 1, HW)
    sum_e2 = jnp.sum(embed_w ** 2, axis=1, keepdims=True)

    ind, diff = pl.pallas_call(
        _vq_body,
        grid=(B,),
        in_specs=[
            pl.BlockSpec((1, C, HW), lambda b: (b, 0, 0)),
            pl.BlockSpec((D, C), lambda b: (0, 0)),
            pl.BlockSpec((D, 1), lambda b: (0, 0)),
            pl.BlockSpec((1, 1, HW), lambda b: (b, 0, 0)),
            pl.BlockSpec((NE, D), lambda b: (0, 0)),
            pl.BlockSpec((NE, 1), lambda b: (0, 0)),
        ],
        out_specs=[
            pl.BlockSpec((1, 1, HW), lambda b: (b, 0, 0)),
            pl.BlockSpec((1, 1), lambda b: (0, 0)),
        ],
        out_shape=[
            jax.ShapeDtypeStruct((B, 1, HW), jnp.int32),
            jax.ShapeDtypeStruct((1, 1), jnp.float32),
        ],
    )(z3, W_proj, bp, sum_f2, embed_w, sum_e2)

    embed_pad = jnp.pad(embed_w, ((0, 0), (0, 128 - D)))
    zq_rows = _sc_gather(embed_pad, ind.reshape(B * HW), B * HW, 128)[:, :D]
    zq = zq_rows.reshape(B, HW, D).transpose(0, 2, 1).reshape(B, D, H, W)

    return zq, diff.reshape(()), ind.reshape(B, H, W)


# speedup vs baseline: 2.7673x; 1.3477x over previous
import functools

import jax
import jax.numpy as jnp
from jax import lax
from jax.experimental import pallas as pl
from jax.experimental.pallas import tpu as pltpu
from jax.experimental.pallas import tpu_sc as plsc

_K = 1024
_BF = jnp.bfloat16


def _dot(a, b):
    return lax.dot_general(a, b, (((1,), (0,)), ((), ())),
                           preferred_element_type=jnp.float32)


def _proj_body(z_ref, w_ref, b_ref, fl_ref):
    ze = _dot(w_ref[...].astype(_BF), z_ref[0].astype(_BF)) + b_ref[...]
    fl_ref[0] = ze.T


def _vq_body(z_ref, w_ref, b_ref, f2_ref, e_ref, e2c_ref, ind_ref, diff_ref):
    b = pl.program_id(0)
    nb = pl.num_programs(0)
    hw = z_ref.shape[2]
    ne = e_ref.shape[0]
    nchunks = ne // _K

    ze = _dot(w_ref[...].astype(_BF), z_ref[0].astype(_BF)) + b_ref[...]
    a_bf = (ze + ze).astype(_BF)
    f2b = jnp.broadcast_to(f2_ref[0], (8, hw))
    siota = lax.broadcasted_iota(jnp.int32, (8, hw), 0)

    def scan_chunk(i, carry):
        e = e_ref[pl.ds(i * _K, _K), :].astype(_BF)
        mm2 = _dot(e, a_bf)
        run8, idx8 = carry
        for g in range(_K // 8):
            mm2v = mm2[g * 8:(g + 1) * 8]
            e2v = e2c_ref[pl.ds(i * _K + g * 8, 8), :]
            dv = (f2b - mm2v) + e2v
            c = dv < run8
            run8 = jnp.where(c, dv, run8)
            idx8 = jnp.where(c, i * (_K // 8) + g, idx8)
        return run8, idx8

    def close(carry):
        run8, idx8 = carry
        v, j = run8, idx8 * 8 + siota
        for w in (4, 2, 1):
            va, vb = v[:w], v[w:2 * w]
            ja, jb = j[:w], j[w:2 * w]
            c = (vb < va) | ((vb == va) & (jb < ja))
            v = jnp.where(c, vb, va)
            j = jnp.where(c, jb, ja)
        return v, j

    init = (jnp.full((8, hw), jnp.inf, jnp.float32),
            jnp.zeros((8, hw), jnp.int32))
    half = nchunks // 2
    min_a, idx_a = close(lax.fori_loop(0, half, scan_chunk, init))
    min_b, idx_b = close(lax.fori_loop(half, nchunks, scan_chunk, init))
    neg_a = (-min_a).astype(_BF).astype(jnp.float32)
    pick_b = (-min_b) > neg_a
    ind_ref[0] = jnp.where(pick_b, idx_b, idx_a)

    win = jnp.where(pick_b, min_b, min_a)
    sq = jnp.sum(win, axis=1, keepdims=True)

    @pl.when(b == 0)
    def _init():
        diff_ref[...] = jnp.zeros_like(diff_ref)

    diff_ref[...] += sq

    @pl.when(b == nb - 1)
    def _finish():
        n_elem = nb * hw * w_ref.shape[0]
        diff_ref[...] = diff_ref[...] * (12.5 / n_elem)


def _sc_gather(table, idx, n_rows, d):
    info = plsc.get_sparse_core_info()
    nw = info.num_cores * info.num_subcores
    per_w = n_rows // nw
    mesh = plsc.VectorSubcoreMesh(core_axis_name="c", subcore_axis_name="s")

    @functools.partial(
        pl.kernel, mesh=mesh,
        out_type=jax.ShapeDtypeStruct((n_rows, d), jnp.float32),
        scratch_types=[
            pltpu.VMEM((per_w,), jnp.int32),
            pltpu.VMEM((per_w, d), jnp.float32),
            pltpu.SemaphoreType.DMA,
        ],
    )
    def k(table_hbm, idx_hbm, out_hbm, idx_v, rows_v, sem):
        wid = lax.axis_index("s") * info.num_cores + lax.axis_index("c")
        base = wid * per_w
        pltpu.sync_copy(idx_hbm.at[pl.ds(base, per_w)], idx_v)
        pltpu.async_copy(table_hbm.at[idx_v], rows_v, sem).wait()
        pltpu.sync_copy(rows_v, out_hbm.at[pl.ds(base, per_w)])

    return k(table, idx)


def kernel(z, W_proj, b_proj, embed_w):
    B, C, H, W = z.shape
    D = W_proj.shape[0]
    NE = embed_w.shape[0]
    HW = H * W

    z3 = z.reshape(B, C, HW)
    bp = b_proj.reshape(D, 1)

    flatten = pl.pallas_call(
        _proj_body,
        grid=(B,),
        in_specs=[
            pl.BlockSpec((1, C, HW), lambda b: (b, 0, 0)),
            pl.BlockSpec((D, C), lambda b: (0, 0)),
            pl.BlockSpec((D, 1), lambda b: (0, 0)),
        ],
        out_specs=pl.BlockSpec((1, HW, D), lambda b: (b, 0, 0)),
        out_shape=jax.ShapeDtypeStruct((B, HW, D), jnp.float32),
    )(z3, W_proj, bp)

    sum_f2 = jnp.sum(flatten.reshape(B, H, W, D) ** 2,
                     axis=3).reshape(B, 1, HW)
    sum_e2 = jnp.sum(embed_w ** 2, axis=1, keepdims=True)

    ind, diff = pl.pallas_call(
        _vq_body,
        grid=(B,),
        in_specs=[
            pl.BlockSpec((1, C, HW), lambda b: (b, 0, 0)),
            pl.BlockSpec((D, C), lambda b: (0, 0)),
            pl.BlockSpec((D, 1), lambda b: (0, 0)),
            pl.BlockSpec((1, 1, HW), lambda b: (b, 0, 0)),
            pl.BlockSpec((NE, D), lambda b: (0, 0)),
            pl.BlockSpec((NE, 1), lambda b: (0, 0)),
        ],
        out_specs=[
            pl.BlockSpec((1, 1, HW), lambda b: (b, 0, 0)),
            pl.BlockSpec((1, 1), lambda b: (0, 0)),
        ],
        out_shape=[
            jax.ShapeDtypeStruct((B, 1, HW), jnp.int32),
            jax.ShapeDtypeStruct((1, 1), jnp.float32),
        ],
    )(z3, W_proj, bp, sum_f2, embed_w, sum_e2)

    embed_pad = jnp.pad(embed_w, ((0, 0), (0, 128 - D)))
    zq_rows = _sc_gather(embed_pad, ind.reshape(B * HW), B * HW, 128)[:, :D]
    zq = zq_rows.reshape(B, HW, D).transpose(0, 2, 1).reshape(B, D, H, W)

    return zq, diff.reshape(()), ind.reshape(B, H, W)


# speedup vs baseline: 3.0165x; 1.0900x over previous
import functools

import jax
import jax.numpy as jnp
from jax import lax
from jax.experimental import pallas as pl
from jax.experimental.pallas import tpu as pltpu
from jax.experimental.pallas import tpu_sc as plsc

_K = 1024
_BF = jnp.bfloat16


def _dot(a, b):
    return lax.dot_general(a, b, (((1,), (0,)), ((), ())),
                           preferred_element_type=jnp.float32)


def _proj_body(z_ref, w_ref, b_ref, fl_ref):
    ze = _dot(w_ref[...].astype(_BF), z_ref[0].astype(_BF)) + b_ref[...]
    fl_ref[0] = ze.T


def _vq_body(z_ref, w_ref, b_ref, f2_ref, e_ref, e2c_ref, ind_ref, diff_ref):
    b = pl.program_id(0)
    nb = pl.num_programs(0)
    hw = z_ref.shape[2]
    ne = e_ref.shape[0]
    nchunks = ne // _K

    ze = _dot(w_ref[...].astype(_BF), z_ref[0].astype(_BF)) + b_ref[...]
    a_bf = (ze + ze).astype(_BF)
    f2b = jnp.broadcast_to(f2_ref[0], (8, hw))
    siota = lax.broadcasted_iota(jnp.int32, (8, hw), 0)

    def scan_chunk(i, carry):
        e = e_ref[pl.ds(i * _K, _K), :].astype(_BF)
        mm2 = _dot(e, a_bf)
        run8, idx8 = carry
        for g in range(_K // 8):
            mm2v = mm2[g * 8:(g + 1) * 8]
            e2v = e2c_ref[pl.ds(i * _K + g * 8, 8), :]
            dv = (f2b - mm2v) + e2v
            c = dv < run8
            run8 = jnp.where(c, dv, run8)
            idx8 = jnp.where(c, i * (_K // 8) + g, idx8)
        return run8, idx8

    def close(carry):
        run8, idx8 = carry
        v, j = run8, idx8 * 8 + siota
        for w in (4, 2, 1):
            va, vb = v[:w], v[w:2 * w]
            ja, jb = j[:w], j[w:2 * w]
            c = (vb < va) | ((vb == va) & (jb < ja))
            v = jnp.where(c, vb, va)
            j = jnp.where(c, jb, ja)
        return v, j

    init = (jnp.full((8, hw), jnp.inf, jnp.float32),
            jnp.zeros((8, hw), jnp.int32))
    half = nchunks // 2
    min_a, idx_a = close(lax.fori_loop(0, half, scan_chunk, init, unroll=4))
    min_b, idx_b = close(lax.fori_loop(half, nchunks, scan_chunk, init,
                                       unroll=4))
    neg_a = (-min_a).astype(_BF).astype(jnp.float32)
    pick_b = (-min_b) > neg_a
    ind_ref[0] = jnp.where(pick_b, idx_b, idx_a)

    win = jnp.where(pick_b, min_b, min_a)
    sq = jnp.sum(win, axis=1, keepdims=True)

    @pl.when(b == 0)
    def _init():
        diff_ref[...] = jnp.zeros_like(diff_ref)

    diff_ref[...] += sq

    @pl.when(b == nb - 1)
    def _finish():
        n_elem = nb * hw * w_ref.shape[0]
        diff_ref[...] = diff_ref[...] * (12.5 / n_elem)


def _sc_gather(table, idx, n_rows, d):
    info = plsc.get_sparse_core_info()
    nw = info.num_cores * info.num_subcores
    per_w = n_rows // nw
    mesh = plsc.VectorSubcoreMesh(core_axis_name="c", subcore_axis_name="s")

    @functools.partial(
        pl.kernel, mesh=mesh,
        out_type=jax.ShapeDtypeStruct((n_rows, d), jnp.float32),
        scratch_types=[
            pltpu.VMEM((per_w,), jnp.int32),
            pltpu.VMEM((per_w, d), jnp.float32),
            pltpu.SemaphoreType.DMA,
        ],
    )
    def k(table_hbm, idx_hbm, out_hbm, idx_v, rows_v, sem):
        wid = lax.axis_index("s") * info.num_cores + lax.axis_index("c")
        base = wid * per_w
        pltpu.sync_copy(idx_hbm.at[pl.ds(base, per_w)], idx_v)
        pltpu.async_copy(table_hbm.at[idx_v], rows_v, sem).wait()
        pltpu.sync_copy(rows_v, out_hbm.at[pl.ds(base, per_w)])

    return k(table, idx)


def kernel(z, W_proj, b_proj, embed_w):
    B, C, H, W = z.shape
    D = W_proj.shape[0]
    NE = embed_w.shape[0]
    HW = H * W

    z3 = z.reshape(B, C, HW)
    bp = b_proj.reshape(D, 1)

    flatten = pl.pallas_call(
        _proj_body,
        grid=(B,),
        in_specs=[
            pl.BlockSpec((1, C, HW), lambda b: (b, 0, 0)),
            pl.BlockSpec((D, C), lambda b: (0, 0)),
            pl.BlockSpec((D, 1), lambda b: (0, 0)),
        ],
        out_specs=pl.BlockSpec((1, HW, D), lambda b: (b, 0, 0)),
        out_shape=jax.ShapeDtypeStruct((B, HW, D), jnp.float32),
    )(z3, W_proj, bp)

    sum_f2 = jnp.sum(flatten.reshape(B, H, W, D) ** 2,
                     axis=3).reshape(B, 1, HW)
    sum_e2 = jnp.sum(embed_w ** 2, axis=1, keepdims=True)

    ind, diff = pl.pallas_call(
        _vq_body,
        grid=(B,),
        in_specs=[
            pl.BlockSpec((1, C, HW), lambda b: (b, 0, 0)),
            pl.BlockSpec((D, C), lambda b: (0, 0)),
            pl.BlockSpec((D, 1), lambda b: (0, 0)),
            pl.BlockSpec((1, 1, HW), lambda b: (b, 0, 0)),
            pl.BlockSpec((NE, D), lambda b: (0, 0)),
            pl.BlockSpec((NE, 1), lambda b: (0, 0)),
        ],
        out_specs=[
            pl.BlockSpec((1, 1, HW), lambda b: (b, 0, 0)),
            pl.BlockSpec((1, 1), lambda b: (0, 0)),
        ],
        out_shape=[
            jax.ShapeDtypeStruct((B, 1, HW), jnp.int32),
            jax.ShapeDtypeStruct((1, 1), jnp.float32),
        ],
    )(z3, W_proj, bp, sum_f2, embed_w, sum_e2)

    embed_pad = jnp.pad(embed_w, ((0, 0), (0, 128 - D)))
    zq_rows = _sc_gather(embed_pad, ind.reshape(B * HW), B * HW, 128)[:, :D]
    zq = zq_rows.reshape(B, HW, D).transpose(0, 2, 1).reshape(B, D, H, W)

    return zq, diff.reshape(()), ind.reshape(B, H, W)
